# baseline (speedup 1.0000x reference)
import functools

import jax
import jax.numpy as jnp
from jax import lax
from jax.experimental import pallas as pl
from jax.experimental.pallas import tpu as pltpu, tpu_sc as plsc
from jax.scipy.linalg import block_diag

NC, NS, L = 2, 16, 16
NW = NC * NS
C = 128
K = 4
DP = 32

_SC_PARAMS = pltpu.CompilerParams(
    needs_layout_passes=False, use_tc_tiling_on_sc=False
)
_MESH = plsc.VectorSubcoreMesh(core_axis_name="c", subcore_axis_name="s")


def _sc_deg(col_r, w_r, n_pad):
    T = col_r.shape[1]
    rows_per_tile = n_pad // NS

    @functools.partial(
        pl.kernel,
        mesh=_MESH,
        out_type=jax.ShapeDtypeStruct((NC, n_pad), jnp.float32),
        scratch_types=[
            pltpu.VMEM((T, C), jnp.int32),
            pltpu.VMEM((T, C), jnp.float32),
            pltpu.VMEM((rows_per_tile,), jnp.float32),
            pltpu.VMEM_SHARED((n_pad,), jnp.float32),
            pltpu.SemaphoreType.DMA,
        ],
        compiler_params=_SC_PARAMS,
    )
    def k(col_hbm, w_hbm, out_hbm, col_v, w_v, zbuf, acc_sh, sem):
        c = lax.axis_index("c")
        s = lax.axis_index("s")
        wid = s * NC + c
        pltpu.sync_copy(col_hbm.at[wid], col_v)
        pltpu.sync_copy(w_hbm.at[wid], w_v)

        @pl.loop(0, rows_per_tile // L)
        def _zero(i):
            zbuf[pl.ds(i * L, L)] = jnp.zeros((L,), jnp.float32)

        pltpu.sync_copy(zbuf, acc_sh.at[pl.ds(s * rows_per_tile, rows_per_tile)])
        plsc.subcore_barrier()

        @pl.loop(0, T)
        def _fire(j):
            pltpu.async_copy(w_v.at[j], acc_sh.at[col_v.at[j]], sem, add=True)

        @pl.loop(0, T)
        def _drain(j):
            pltpu.make_async_copy(w_v.at[j], acc_sh.at[col_v.at[j]], sem).wait()

        plsc.subcore_barrier()
        pltpu.sync_copy(
            acc_sh.at[pl.ds(s * rows_per_tile, rows_per_tile)],
            out_hbm.at[c, pl.ds(s * rows_per_tile, rows_per_tile)],
        )

    return k(col_r, w_r)


def _sc_norm(dinv_lin, row_r, col_r, w_r):
    T = row_r.shape[1]

    @functools.partial(
        pl.kernel,
        mesh=_MESH,
        out_type=jax.ShapeDtypeStruct((NW, T, C), jnp.float32),
        scratch_types=[
            pltpu.VMEM((T, C), jnp.int32),
            pltpu.VMEM((T, C), jnp.int32),
            pltpu.VMEM((T, C), jnp.float32),
            pltpu.VMEM((T, C), jnp.float32),
            pltpu.VMEM((K, C), jnp.float32),
            pltpu.VMEM((K, C), jnp.float32),
        ] + [pltpu.SemaphoreType.DMA] * K,
        compiler_params=_SC_PARAMS,
    )
    def k(dinv_hbm, row_hbm, col_hbm, w_hbm, out_hbm,
          row_v, col_v, w_v, nrm_v, dr_v, dc_v, *sems):
        c = lax.axis_index("c")
        s = lax.axis_index("s")
        wid = s * NC + c
        pltpu.sync_copy(row_hbm.at[wid], row_v)
        pltpu.sync_copy(col_hbm.at[wid], col_v)
        pltpu.sync_copy(w_hbm.at[wid], w_v)

        for b in range(K):
            pltpu.async_copy(dinv_hbm.at[row_v.at[b]], dr_v.at[b], sems[b])
            pltpu.async_copy(dinv_hbm.at[col_v.at[b]], dc_v.at[b], sems[b])

        @pl.loop(0, T // K)
        def _ring(gi):
            for b in range(K):
                j = gi * K + b
                pltpu.make_async_copy(
                    dinv_hbm.at[row_v.at[j]], dr_v.at[b], sems[b]
                ).wait()
                pltpu.make_async_copy(
                    dinv_hbm.at[col_v.at[j]], dc_v.at[b], sems[b]
                ).wait()

                @pl.loop(0, C // L)
                def _mul(q):
                    sl = pl.ds(q * L, L)
                    nrm_v[j, sl] = w_v[j, sl] * dr_v[b, sl] * dc_v[b, sl]

                @pl.when(j + K < T)
                def _refill():
                    pltpu.async_copy(
                        dinv_hbm.at[row_v.at[j + K]], dr_v.at[b], sems[b]
                    )
                    pltpu.async_copy(
                        dinv_hbm.at[col_v.at[j + K]], dc_v.at[b], sems[b]
                    )

        pltpu.sync_copy(nrm_v, out_hbm.at[wid])

    return k(dinv_lin, row_r, col_r, w_r)


def _sc_agg(hd, row_r, col_r, nrm_r, n_pad):
    T = row_r.shape[1]
    rows_per_tile = n_pad // NS
    nz = rows_per_tile // C
    R = 2 * K

    @functools.partial(
        pl.kernel,
        mesh=_MESH,
        out_type=jax.ShapeDtypeStruct((NC, n_pad, DP), jnp.float32),
        scratch_types=[
            pltpu.VMEM((T, C), jnp.int32),
            pltpu.VMEM((T, C), jnp.int32),
            pltpu.VMEM((T, C), jnp.float32),
            pltpu.VMEM((R, C, DP), jnp.float32),
            pltpu.VMEM_SHARED((n_pad, DP), jnp.float32),
        ] + [pltpu.SemaphoreType.DMA] * (2 * R),
        compiler_params=_SC_PARAMS,
    )
    def k(hd_hbm, row_hbm, col_hbm, w_hbm, out_hbm,
          row_v, col_v, w_v, msg_v, acc_sh, *sems):
        gsem = sems[:R]
        ssem = sems[R:]
        c = lax.axis_index("c")
        s = lax.axis_index("s")
        wid = s * NC + c
        pltpu.sync_copy(row_hbm.at[wid], row_v)
        pltpu.sync_copy(col_hbm.at[wid], col_v)
        pltpu.sync_copy(w_hbm.at[wid], w_v)

        zb = msg_v.at[0]

        @pl.loop(0, (C * DP) // L)
        def _zero(i):
            r = i // (DP // L)
            kk = i % (DP // L)
            zb[r, pl.ds(kk * L, L)] = jnp.zeros((L,), jnp.float32)

        @pl.loop(0, nz)
        def _zcopy(kz):
            pltpu.sync_copy(zb, acc_sh.at[pl.ds(s * rows_per_tile + kz * C, C)])

        plsc.subcore_barrier()

        for b in range(K):
            pltpu.async_copy(hd_hbm.at[row_v.at[b]], msg_v.at[b], gsem[b])

        def scale_chunk(mb, j):
            @pl.loop(0, C // 16)
            def _grp(q):
                wrow = w_v[j, pl.ds(q * 16, 16)]
                for l in range(16):
                    wv = jnp.full((L,), wrow[l], jnp.float32)
                    e = q * 16 + l
                    for kk in range(DP // L):
                        sl = pl.ds(kk * L, L)
                        mb[e, sl] = mb[e, sl] * wv

        @pl.loop(0, T // R)
        def _ring(gi):
            for v in range(R):
                j = gi * R + v
                mb = msg_v.at[v]
                pltpu.make_async_copy(hd_hbm.at[row_v.at[j]], mb, gsem[v]).wait()
                scale_chunk(mb, j)
                pltpu.async_copy(mb, acc_sh.at[col_v.at[j]], ssem[v], add=True)

                jg = j + K
                bg = (v + K) % R
                mg = msg_v.at[bg]

                @pl.when(jg < T)
                def _refill():
                    @pl.when(jg >= R)
                    def _drain_prev_scatter():
                        pltpu.make_async_copy(
                            mg, acc_sh.at[col_v.at[jg - R]], ssem[bg]
                        ).wait()

                    pltpu.async_copy(hd_hbm.at[row_v.at[jg]], mg, gsem[bg])

        for b in range(R):
            pltpu.make_async_copy(
                msg_v.at[b], acc_sh.at[col_v.at[T - R + b]], ssem[b]
            ).wait()

        plsc.subcore_barrier()
        pltpu.sync_copy(
            acc_sh.at[pl.ds(s * rows_per_tile, rows_per_tile)],
            out_hbm.at[c, pl.ds(s * rows_per_tile, rows_per_tile)],
        )

    return k(hd, row_r, col_r, nrm_r)


def _tc_dinv(degp2, n_pad):
    half = n_pad // 128

    def body(p_ref, o_ref):
        a = p_ref[...]
        o_ref[...] = lax.rsqrt(a[:half] + a[half:])

    return pl.pallas_call(
        body, out_shape=jax.ShapeDtypeStruct((half, 128), jnp.float32)
    )(degp2)


def _tc_pre1(x, Wp):
    n, d_in = x.shape

    def body(x_ref, w_ref, o_ref):
        o_ref[...] = jnp.dot(
            x_ref[...], w_ref[...], preferred_element_type=jnp.float32
        )

    return pl.pallas_call(
        body, out_shape=jax.ShapeDtypeStruct((n, DP), jnp.float32)
    )(x, Wp)


def _tc_step(P_pack, BD, bt):
    two_pr = P_pack.shape[0]
    pr = two_pr // 2

    def body(p_ref, w_ref, b_ref, o_ref):
        a = p_ref[...]
        s = a[:pr] + a[pr:]
        x = jnp.maximum(s + b_ref[...], 0.0)
        o_ref[...] = jnp.dot(x, w_ref[...], preferred_element_type=jnp.float32)

    return pl.pallas_call(
        body, out_shape=jax.ShapeDtypeStruct((pr, 128), jnp.float32)
    )(P_pack, BD, bt)


def _tc_final(P, Wp, br, n, d_out):
    def body(p_ref, w_ref, b_ref, o_ref):
        s = p_ref[0, :n, :] + p_ref[1, :n, :]
        out = jnp.dot(s, w_ref[...], preferred_element_type=jnp.float32)
        out = out[:, :d_out] + b_ref[...]
        m = jnp.max(out, axis=1, keepdims=True)
        z = out - m
        o_ref[...] = z - jnp.log(jnp.sum(jnp.exp(z), axis=1, keepdims=True))

    return pl.pallas_call(
        body, out_shape=jax.ShapeDtypeStruct((n, d_out), jnp.float32)
    )(P, Wp, br)


def _padw(W, a, b):
    return jnp.pad(W, ((0, a - W.shape[0]), (0, b - W.shape[1])))


def _bd4(Wp):
    return block_diag(Wp, Wp, Wp, Wp)


def _bt4(b):
    return jnp.tile(jnp.pad(b, (0, DP - b.shape[0])), 4)[None, :]


def kernel(features, edges, weights, W1, b1, W2, b2, W3, b3, W4, b4):
    n = features.shape[0]
    e_cnt = edges.shape[1]
    row = edges[0].astype(jnp.int32)
    col = edges[1].astype(jnp.int32)
    w = weights.astype(jnp.float32)

    loop = jnp.arange(n, dtype=jnp.int32)
    e_tot = e_cnt + n
    T = -(-e_tot // (NW * C * 2 * K)) * (2 * K)
    e_pad = NW * T * C
    npad = e_pad - e_tot
    fill = (jnp.arange(npad, dtype=jnp.int32) * 97) % n
    row = jnp.concatenate([row, loop, fill])
    col = jnp.concatenate([col, loop, fill])
    w = jnp.concatenate(
        [w, jnp.ones((n,), jnp.float32), jnp.zeros((npad,), jnp.float32)]
    )
    row_r = row.reshape(NW, T, C)
    col_r = col.reshape(NW, T, C)
    w_r = w.reshape(NW, T, C)

    n_pad = -(-n // (NS * C)) * (NS * C)
    pr = n_pad // 4

    degp = _sc_deg(col_r, w_r, n_pad)
    dinv = _tc_dinv(degp.reshape(2 * n_pad // 128, 128), n_pad).reshape(n_pad)
    nrm_r = _sc_norm(dinv, row_r, col_r, w_r)

    W1p = _padw(W1, 128, DP)
    BD2 = _bd4(_padw(W2, DP, DP))
    BD3 = _bd4(_padw(W3, DP, DP))
    I128 = jnp.eye(128, dtype=jnp.float32)
    W4p = _padw(W4, DP, 48)

    h1 = _tc_pre1(features, W1p)
    P = _sc_agg(h1, row_r, col_r, nrm_r, n_pad)
    hp = _tc_step(P.reshape(2 * pr, 128), BD2, _bt4(b1))
    P = _sc_agg(hp.reshape(n_pad, DP), row_r, col_r, nrm_r, n_pad)
    hp = _tc_step(P.reshape(2 * pr, 128), BD2, _bt4(b2))
    P = _sc_agg(hp.reshape(n_pad, DP), row_r, col_r, nrm_r, n_pad)
    hp = _tc_step(P.reshape(2 * pr, 128), BD3, _bt4(b2))
    P = _sc_agg(hp.reshape(n_pad, DP), row_r, col_r, nrm_r, n_pad)
    xp = _tc_step(P.reshape(2 * pr, 128), I128, _bt4(b3))
    P = _sc_agg(xp.reshape(n_pad, DP), row_r, col_r, nrm_r, n_pad)
    return _tc_final(P, W4p, b4[None, :], n, 40)


# speedup vs baseline: 1.2356x; 1.2356x over previous
import functools

import jax
import jax.numpy as jnp
from jax import lax
from jax.experimental import pallas as pl
from jax.experimental.pallas import tpu as pltpu, tpu_sc as plsc
from jax.scipy.linalg import block_diag

NC, NS, L = 2, 16, 16
NW = NC * NS
C = 128
K = 4
DP = 32

_SC_PARAMS = pltpu.CompilerParams(
    needs_layout_passes=False, use_tc_tiling_on_sc=False
)
_MESH = plsc.VectorSubcoreMesh(core_axis_name="c", subcore_axis_name="s")


def _sc_deg(col_r, w_r, n_pad):
    T = col_r.shape[1]
    rows_per_tile = n_pad // NS

    @functools.partial(
        pl.kernel,
        mesh=_MESH,
        out_type=jax.ShapeDtypeStruct((NC, n_pad), jnp.float32),
        scratch_types=[
            pltpu.VMEM((T, C), jnp.int32),
            pltpu.VMEM((T, C), jnp.float32),
            pltpu.VMEM((rows_per_tile,), jnp.float32),
            pltpu.VMEM_SHARED((n_pad,), jnp.float32),
            pltpu.SemaphoreType.DMA,
        ],
        compiler_params=_SC_PARAMS,
    )
    def k(col_hbm, w_hbm, out_hbm, col_v, w_v, zbuf, acc_sh, sem):
        c = lax.axis_index("c")
        s = lax.axis_index("s")
        wid = s * NC + c
        pltpu.sync_copy(col_hbm.at[wid], col_v)
        pltpu.sync_copy(w_hbm.at[wid], w_v)

        @pl.loop(0, rows_per_tile // L)
        def _zero(i):
            zbuf[pl.ds(i * L, L)] = jnp.zeros((L,), jnp.float32)

        pltpu.sync_copy(zbuf, acc_sh.at[pl.ds(s * rows_per_tile, rows_per_tile)])
        plsc.subcore_barrier()

        @pl.loop(0, T)
        def _fire(j):
            pltpu.async_copy(w_v.at[j], acc_sh.at[col_v.at[j]], sem, add=True)

        @pl.loop(0, T)
        def _drain(j):
            pltpu.make_async_copy(w_v.at[j], acc_sh.at[col_v.at[j]], sem).wait()

        plsc.subcore_barrier()
        pltpu.sync_copy(
            acc_sh.at[pl.ds(s * rows_per_tile, rows_per_tile)],
            out_hbm.at[c, pl.ds(s * rows_per_tile, rows_per_tile)],
        )

    return k(col_r, w_r)


def _sc_norm(dinv_lin, row_r, col_r, w_r):
    T = row_r.shape[1]
    n_pad = dinv_lin.shape[0]

    @functools.partial(
        pl.kernel,
        mesh=_MESH,
        out_type=jax.ShapeDtypeStruct((NW, T, C), jnp.float32),
        scratch_types=[
            pltpu.VMEM((T, C), jnp.int32),
            pltpu.VMEM((T, C), jnp.int32),
            pltpu.VMEM((T, C), jnp.float32),
            pltpu.VMEM((T, C), jnp.float32),
            pltpu.VMEM((n_pad,), jnp.float32),
        ],
        compiler_params=_SC_PARAMS,
    )
    def k(dinv_hbm, row_hbm, col_hbm, w_hbm, out_hbm,
          row_v, col_v, w_v, nrm_v, dinv_v):
        c = lax.axis_index("c")
        s = lax.axis_index("s")
        wid = s * NC + c
        pltpu.sync_copy(row_hbm.at[wid], row_v)
        pltpu.sync_copy(col_hbm.at[wid], col_v)
        pltpu.sync_copy(w_hbm.at[wid], w_v)
        pltpu.sync_copy(dinv_hbm, dinv_v)

        @pl.loop(0, T * (C // L))
        def _mul(i):
            j = i // (C // L)
            q = i % (C // L)
            sl = pl.ds(q * L, L)
            dr = plsc.load_gather(dinv_v, [row_v[j, sl]])
            dc = plsc.load_gather(dinv_v, [col_v[j, sl]])
            nrm_v[j, sl] = w_v[j, sl] * dr * dc

        pltpu.sync_copy(nrm_v, out_hbm.at[wid])

    return k(dinv_lin, row_r, col_r, w_r)


def _sc_agg(hd, row_r, col_r, nrm_r, n_pad):
    T = row_r.shape[1]
    rows_per_tile = n_pad // NS
    nz = rows_per_tile // C
    R = 2 * K

    @functools.partial(
        pl.kernel,
        mesh=_MESH,
        out_type=jax.ShapeDtypeStruct((NC, n_pad, DP), jnp.float32),
        scratch_types=[
            pltpu.VMEM((T, C), jnp.int32),
            pltpu.VMEM((T, C), jnp.int32),
            pltpu.VMEM((T, C), jnp.float32),
            pltpu.VMEM((R, C, DP), jnp.float32),
            pltpu.VMEM_SHARED((n_pad, DP), jnp.float32),
        ] + [pltpu.SemaphoreType.DMA] * (2 * R),
        compiler_params=_SC_PARAMS,
    )
    def k(hd_hbm, row_hbm, col_hbm, w_hbm, out_hbm,
          row_v, col_v, w_v, msg_v, acc_sh, *sems):
        gsem = sems[:R]
        ssem = sems[R:]
        c = lax.axis_index("c")
        s = lax.axis_index("s")
        wid = s * NC + c
        pltpu.sync_copy(row_hbm.at[wid], row_v)
        pltpu.sync_copy(col_hbm.at[wid], col_v)
        pltpu.sync_copy(w_hbm.at[wid], w_v)

        zb = msg_v.at[0]

        @pl.loop(0, (C * DP) // L)
        def _zero(i):
            r = i // (DP // L)
            kk = i % (DP // L)
            zb[r, pl.ds(kk * L, L)] = jnp.zeros((L,), jnp.float32)

        @pl.loop(0, nz)
        def _zcopy(kz):
            pltpu.sync_copy(zb, acc_sh.at[pl.ds(s * rows_per_tile + kz * C, C)])

        plsc.subcore_barrier()

        for b in range(K):
            pltpu.async_copy(hd_hbm.at[row_v.at[b]], msg_v.at[b], gsem[b])

        def scale_chunk(mb, j):
            @pl.loop(0, C // 16)
            def _grp(q):
                wrow = w_v[j, pl.ds(q * 16, 16)]
                for l in range(16):
                    wv = jnp.full((L,), wrow[l], jnp.float32)
                    e = q * 16 + l
                    for kk in range(DP // L):
                        sl = pl.ds(kk * L, L)
                        mb[e, sl] = mb[e, sl] * wv

        @pl.loop(0, T // R)
        def _ring(gi):
            for v in range(R):
                j = gi * R + v
                mb = msg_v.at[v]
                pltpu.make_async_copy(hd_hbm.at[row_v.at[j]], mb, gsem[v]).wait()
                scale_chunk(mb, j)
                pltpu.async_copy(mb, acc_sh.at[col_v.at[j]], ssem[v], add=True)

                jg = j + K
                bg = (v + K) % R
                mg = msg_v.at[bg]

                @pl.when(jg < T)
                def _refill():
                    @pl.when(jg >= R)
                    def _drain_prev_scatter():
                        pltpu.make_async_copy(
                            mg, acc_sh.at[col_v.at[jg - R]], ssem[bg]
                        ).wait()

                    pltpu.async_copy(hd_hbm.at[row_v.at[jg]], mg, gsem[bg])

        for b in range(R):
            pltpu.make_async_copy(
                msg_v.at[b], acc_sh.at[col_v.at[T - R + b]], ssem[b]
            ).wait()

        plsc.subcore_barrier()
        pltpu.sync_copy(
            acc_sh.at[pl.ds(s * rows_per_tile, rows_per_tile)],
            out_hbm.at[c, pl.ds(s * rows_per_tile, rows_per_tile)],
        )

    return k(hd, row_r, col_r, nrm_r)


def _tc_dinv(degp2, n_pad):
    half = n_pad // 128

    def body(p_ref, o_ref):
        a = p_ref[...]
        o_ref[...] = lax.rsqrt(a[:half] + a[half:])

    return pl.pallas_call(
        body, out_shape=jax.ShapeDtypeStruct((half, 128), jnp.float32)
    )(degp2)


def _tc_pre1(x, Wp):
    n, d_in = x.shape

    def body(x_ref, w_ref, o_ref):
        o_ref[...] = jnp.dot(
            x_ref[...], w_ref[...], preferred_element_type=jnp.float32
        )

    return pl.pallas_call(
        body, out_shape=jax.ShapeDtypeStruct((n, DP), jnp.float32)
    )(x, Wp)


def _tc_step(P_pack, BD, bt):
    two_pr = P_pack.shape[0]
    pr = two_pr // 2

    def body(p_ref, w_ref, b_ref, o_ref):
        a = p_ref[...]
        s = a[:pr] + a[pr:]
        x = jnp.maximum(s + b_ref[...], 0.0)
        o_ref[...] = jnp.dot(x, w_ref[...], preferred_element_type=jnp.float32)

    return pl.pallas_call(
        body, out_shape=jax.ShapeDtypeStruct((pr, 128), jnp.float32)
    )(P_pack, BD, bt)


def _tc_final(P, Wp, br, n, d_out):
    def body(p_ref, w_ref, b_ref, o_ref):
        s = p_ref[0, :n, :] + p_ref[1, :n, :]
        out = jnp.dot(s, w_ref[...], preferred_element_type=jnp.float32)
        out = out[:, :d_out] + b_ref[...]
        m = jnp.max(out, axis=1, keepdims=True)
        z = out - m
        o_ref[...] = z - jnp.log(jnp.sum(jnp.exp(z), axis=1, keepdims=True))

    return pl.pallas_call(
        body, out_shape=jax.ShapeDtypeStruct((n, d_out), jnp.float32)
    )(P, Wp, br)


def _padw(W, a, b):
    return jnp.pad(W, ((0, a - W.shape[0]), (0, b - W.shape[1])))


def _bd4(Wp):
    return block_diag(Wp, Wp, Wp, Wp)


def _bt4(b):
    return jnp.tile(jnp.pad(b, (0, DP - b.shape[0])), 4)[None, :]


def kernel(features, edges, weights, W1, b1, W2, b2, W3, b3, W4, b4):
    n = features.shape[0]
    e_cnt = edges.shape[1]
    row = edges[0].astype(jnp.int32)
    col = edges[1].astype(jnp.int32)
    w = weights.astype(jnp.float32)

    loop = jnp.arange(n, dtype=jnp.int32)
    e_tot = e_cnt + n
    T = -(-e_tot // (NW * C * 2 * K)) * (2 * K)
    e_pad = NW * T * C
    npad = e_pad - e_tot
    fill = (jnp.arange(npad, dtype=jnp.int32) * 97) % n
    row = jnp.concatenate([row, loop, fill])
    col = jnp.concatenate([col, loop, fill])
    w = jnp.concatenate(
        [w, jnp.ones((n,), jnp.float32), jnp.zeros((npad,), jnp.float32)]
    )
    row_r = row.reshape(NW, T, C)
    col_r = col.reshape(NW, T, C)
    w_r = w.reshape(NW, T, C)

    n_pad = -(-n // (NS * C)) * (NS * C)
    pr = n_pad // 4

    degp = _sc_deg(col_r, w_r, n_pad)
    dinv = _tc_dinv(degp.reshape(2 * n_pad // 128, 128), n_pad).reshape(n_pad)
    nrm_r = _sc_norm(dinv, row_r, col_r, w_r)

    W1p = _padw(W1, 128, DP)
    BD2 = _bd4(_padw(W2, DP, DP))
    BD3 = _bd4(_padw(W3, DP, DP))
    I128 = jnp.eye(128, dtype=jnp.float32)
    W4p = _padw(W4, DP, 48)

    h1 = _tc_pre1(features, W1p)
    P = _sc_agg(h1, row_r, col_r, nrm_r, n_pad)
    hp = _tc_step(P.reshape(2 * pr, 128), BD2, _bt4(b1))
    P = _sc_agg(hp.reshape(n_pad, DP), row_r, col_r, nrm_r, n_pad)
    hp = _tc_step(P.reshape(2 * pr, 128), BD2, _bt4(b2))
    P = _sc_agg(hp.reshape(n_pad, DP), row_r, col_r, nrm_r, n_pad)
    hp = _tc_step(P.reshape(2 * pr, 128), BD3, _bt4(b2))
    P = _sc_agg(hp.reshape(n_pad, DP), row_r, col_r, nrm_r, n_pad)
    xp = _tc_step(P.reshape(2 * pr, 128), I128, _bt4(b3))
    P = _sc_agg(xp.reshape(n_pad, DP), row_r, col_r, nrm_r, n_pad)
    return _tc_final(P, W4p, b4[None, :], n, 40)


# speedup vs baseline: 1.2920x; 1.0456x over previous
import functools

import jax
import jax.numpy as jnp
from jax import lax
from jax.experimental import pallas as pl
from jax.experimental.pallas import tpu as pltpu, tpu_sc as plsc
from jax.scipy.linalg import block_diag

NC, NS, L = 2, 16, 16
NW = NC * NS
C = 128
K = 4
DP = 32

_SC_PARAMS = pltpu.CompilerParams(
    needs_layout_passes=False, use_tc_tiling_on_sc=False
)
_MESH = plsc.VectorSubcoreMesh(core_axis_name="c", subcore_axis_name="s")


def _sc_deg(col_r, w_r, n_pad):
    T = col_r.shape[1]
    rows_per_tile = n_pad // NS

    @functools.partial(
        pl.kernel,
        mesh=_MESH,
        out_type=jax.ShapeDtypeStruct((NC, n_pad), jnp.float32),
        scratch_types=[
            pltpu.VMEM((T, C), jnp.int32),
            pltpu.VMEM((T, C), jnp.float32),
            pltpu.VMEM((rows_per_tile,), jnp.float32),
            pltpu.VMEM_SHARED((n_pad,), jnp.float32),
            pltpu.SemaphoreType.DMA,
            pltpu.SemaphoreType.DMA,
            pltpu.SemaphoreType.DMA,
        ],
        compiler_params=_SC_PARAMS,
    )
    def k(col_hbm, w_hbm, out_hbm, col_v, w_v, zbuf, acc_sh, sem, sa, sb):
        c = lax.axis_index("c")
        s = lax.axis_index("s")
        wid = s * NC + c
        pltpu.async_copy(col_hbm.at[wid], col_v, sa)
        pltpu.async_copy(w_hbm.at[wid], w_v, sb)

        @pl.loop(0, rows_per_tile // L)
        def _zero(i):
            zbuf[pl.ds(i * L, L)] = jnp.zeros((L,), jnp.float32)

        pltpu.make_async_copy(col_hbm.at[wid], col_v, sa).wait()
        pltpu.make_async_copy(w_hbm.at[wid], w_v, sb).wait()
        pltpu.sync_copy(zbuf, acc_sh.at[pl.ds(s * rows_per_tile, rows_per_tile)])
        plsc.subcore_barrier()

        @pl.loop(0, T)
        def _fire(j):
            pltpu.async_copy(w_v.at[j], acc_sh.at[col_v.at[j]], sem, add=True)

        @pl.loop(0, T)
        def _drain(j):
            pltpu.make_async_copy(w_v.at[j], acc_sh.at[col_v.at[j]], sem).wait()

        plsc.subcore_barrier()
        pltpu.sync_copy(
            acc_sh.at[pl.ds(s * rows_per_tile, rows_per_tile)],
            out_hbm.at[c, pl.ds(s * rows_per_tile, rows_per_tile)],
        )

    return k(col_r, w_r)


def _sc_norm(dinv_lin, row_r, col_r, w_r):
    T = row_r.shape[1]
    n_pad = dinv_lin.shape[0]

    @functools.partial(
        pl.kernel,
        mesh=_MESH,
        out_type=jax.ShapeDtypeStruct((NW, T, C), jnp.float32),
        scratch_types=[
            pltpu.VMEM((T, C), jnp.int32),
            pltpu.VMEM((T, C), jnp.int32),
            pltpu.VMEM((T, C), jnp.float32),
            pltpu.VMEM((T, C), jnp.float32),
            pltpu.VMEM((n_pad,), jnp.float32),
        ] + [pltpu.SemaphoreType.DMA] * 4,
        compiler_params=_SC_PARAMS,
    )
    def k(dinv_hbm, row_hbm, col_hbm, w_hbm, out_hbm,
          row_v, col_v, w_v, nrm_v, dinv_v, *sems):
        c = lax.axis_index("c")
        s = lax.axis_index("s")
        wid = s * NC + c
        pltpu.async_copy(row_hbm.at[wid], row_v, sems[0])
        pltpu.async_copy(col_hbm.at[wid], col_v, sems[1])
        pltpu.async_copy(w_hbm.at[wid], w_v, sems[2])
        pltpu.async_copy(dinv_hbm, dinv_v, sems[3])
        pltpu.make_async_copy(row_hbm.at[wid], row_v, sems[0]).wait()
        pltpu.make_async_copy(col_hbm.at[wid], col_v, sems[1]).wait()
        pltpu.make_async_copy(w_hbm.at[wid], w_v, sems[2]).wait()
        pltpu.make_async_copy(dinv_hbm, dinv_v, sems[3]).wait()

        @pl.loop(0, T * (C // L))
        def _mul(i):
            j = i // (C // L)
            q = i % (C // L)
            sl = pl.ds(q * L, L)
            dr = plsc.load_gather(dinv_v, [row_v[j, sl]])
            dc = plsc.load_gather(dinv_v, [col_v[j, sl]])
            nrm_v[j, sl] = w_v[j, sl] * dr * dc

        pltpu.sync_copy(nrm_v, out_hbm.at[wid])

    return k(dinv_lin, row_r, col_r, w_r)


def _sc_agg(hd, row_r, col_r, nrm_r, n_pad):
    T = row_r.shape[1]
    rows_per_tile = n_pad // NS
    nz = rows_per_tile // C
    R = 2 * K

    @functools.partial(
        pl.kernel,
        mesh=_MESH,
        out_type=jax.ShapeDtypeStruct((NC, n_pad, DP), jnp.float32),
        scratch_types=[
            pltpu.VMEM((T, C), jnp.int32),
            pltpu.VMEM((T, C), jnp.int32),
            pltpu.VMEM((T, C), jnp.float32),
            pltpu.VMEM((R, C, DP), jnp.float32),
            pltpu.VMEM_SHARED((n_pad, DP), jnp.float32),
        ] + [pltpu.SemaphoreType.DMA] * (2 * R),
        compiler_params=_SC_PARAMS,
    )
    def k(hd_hbm, row_hbm, col_hbm, w_hbm, out_hbm,
          row_v, col_v, w_v, msg_v, acc_sh, *sems):
        gsem = sems[:R]
        ssem = sems[R:]
        c = lax.axis_index("c")
        s = lax.axis_index("s")
        wid = s * NC + c
        pltpu.async_copy(row_hbm.at[wid], row_v, gsem[0])
        pltpu.async_copy(col_hbm.at[wid], col_v, gsem[1])
        pltpu.async_copy(w_hbm.at[wid], w_v, gsem[2])

        zb = msg_v.at[0]

        @pl.loop(0, (C * DP) // L)
        def _zero(i):
            r = i // (DP // L)
            kk = i % (DP // L)
            zb[r, pl.ds(kk * L, L)] = jnp.zeros((L,), jnp.float32)

        for kz in range(nz):
            pltpu.async_copy(
                zb, acc_sh.at[pl.ds(s * rows_per_tile + kz * C, C)], ssem[kz]
            )
        pltpu.make_async_copy(row_hbm.at[wid], row_v, gsem[0]).wait()
        pltpu.make_async_copy(col_hbm.at[wid], col_v, gsem[1]).wait()
        pltpu.make_async_copy(w_hbm.at[wid], w_v, gsem[2]).wait()
        for kz in range(nz):
            pltpu.make_async_copy(
                zb, acc_sh.at[pl.ds(s * rows_per_tile + kz * C, C)], ssem[kz]
            ).wait()

        plsc.subcore_barrier()

        for b in range(K):
            pltpu.async_copy(hd_hbm.at[row_v.at[b]], msg_v.at[b], gsem[b])

        def scale_chunk(mb, j):
            @pl.loop(0, C // 16)
            def _grp(q):
                wrow = w_v[j, pl.ds(q * 16, 16)]
                for l in range(16):
                    wv = jnp.full((L,), wrow[l], jnp.float32)
                    e = q * 16 + l
                    for kk in range(DP // L):
                        sl = pl.ds(kk * L, L)
                        mb[e, sl] = mb[e, sl] * wv

        @pl.loop(0, T // R)
        def _ring(gi):
            for v in range(R):
                j = gi * R + v
                mb = msg_v.at[v]
                pltpu.make_async_copy(hd_hbm.at[row_v.at[j]], mb, gsem[v]).wait()
                scale_chunk(mb, j)
                pltpu.async_copy(mb, acc_sh.at[col_v.at[j]], ssem[v], add=True)

                jg = j + K
                bg = (v + K) % R
                mg = msg_v.at[bg]

                @pl.when(jg < T)
                def _refill():
                    @pl.when(jg >= R)
                    def _drain_prev_scatter():
                        pltpu.make_async_copy(
                            mg, acc_sh.at[col_v.at[jg - R]], ssem[bg]
                        ).wait()

                    pltpu.async_copy(hd_hbm.at[row_v.at[jg]], mg, gsem[bg])

        for b in range(R):
            pltpu.make_async_copy(
                msg_v.at[b], acc_sh.at[col_v.at[T - R + b]], ssem[b]
            ).wait()

        plsc.subcore_barrier()
        pltpu.sync_copy(
            acc_sh.at[pl.ds(s * rows_per_tile, rows_per_tile)],
            out_hbm.at[c, pl.ds(s * rows_per_tile, rows_per_tile)],
        )

    return k(hd, row_r, col_r, nrm_r)


def _tc_dinv(degp2, n_pad):
    half = n_pad // 128

    def body(p_ref, o_ref):
        a = p_ref[...]
        o_ref[...] = lax.rsqrt(a[:half] + a[half:])

    return pl.pallas_call(
        body, out_shape=jax.ShapeDtypeStruct((half, 128), jnp.float32)
    )(degp2)


def _tc_pre1(x, Wp):
    n, d_in = x.shape

    def body(x_ref, w_ref, o_ref):
        o_ref[...] = jnp.dot(
            x_ref[...], w_ref[...], preferred_element_type=jnp.float32
        )

    return pl.pallas_call(
        body, out_shape=jax.ShapeDtypeStruct((n, DP), jnp.float32)
    )(x, Wp)


def _tc_step(P_pack, BD, bt):
    two_pr = P_pack.shape[0]
    pr = two_pr // 2

    def body(p_ref, w_ref, b_ref, o_ref):
        a = p_ref[...]
        s = a[:pr] + a[pr:]
        x = jnp.maximum(s + b_ref[...], 0.0)
        o_ref[...] = jnp.dot(x, w_ref[...], preferred_element_type=jnp.float32)

    return pl.pallas_call(
        body, out_shape=jax.ShapeDtypeStruct((pr, 128), jnp.float32)
    )(P_pack, BD, bt)


def _tc_final(P, Wp, br, n, d_out):
    def body(p_ref, w_ref, b_ref, o_ref):
        s = p_ref[0, :n, :] + p_ref[1, :n, :]
        out = jnp.dot(s, w_ref[...], preferred_element_type=jnp.float32)
        out = out[:, :d_out] + b_ref[...]
        m = jnp.max(out, axis=1, keepdims=True)
        z = out - m
        o_ref[...] = z - jnp.log(jnp.sum(jnp.exp(z), axis=1, keepdims=True))

    return pl.pallas_call(
        body, out_shape=jax.ShapeDtypeStruct((n, d_out), jnp.float32)
    )(P, Wp, br)


def _padw(W, a, b):
    return jnp.pad(W, ((0, a - W.shape[0]), (0, b - W.shape[1])))


def _bd4(Wp):
    return block_diag(Wp, Wp, Wp, Wp)


def _bt4(b):
    return jnp.tile(jnp.pad(b, (0, DP - b.shape[0])), 4)[None, :]


def kernel(features, edges, weights, W1, b1, W2, b2, W3, b3, W4, b4):
    n = features.shape[0]
    e_cnt = edges.shape[1]
    row = edges[0].astype(jnp.int32)
    col = edges[1].astype(jnp.int32)
    w = weights.astype(jnp.float32)

    loop = jnp.arange(n, dtype=jnp.int32)
    e_tot = e_cnt + n
    T = -(-e_tot // (NW * C * 2 * K)) * (2 * K)
    e_pad = NW * T * C
    npad = e_pad - e_tot
    fill = (jnp.arange(npad, dtype=jnp.int32) * 97) % n
    row = jnp.concatenate([row, loop, fill])
    col = jnp.concatenate([col, loop, fill])
    w = jnp.concatenate(
        [w, jnp.ones((n,), jnp.float32), jnp.zeros((npad,), jnp.float32)]
    )
    row_r = row.reshape(NW, T, C)
    col_r = col.reshape(NW, T, C)
    w_r = w.reshape(NW, T, C)

    n_pad = -(-n // (NS * C)) * (NS * C)
    pr = n_pad // 4

    degp = _sc_deg(col_r, w_r, n_pad)
    dinv = _tc_dinv(degp.reshape(2 * n_pad // 128, 128), n_pad).reshape(n_pad)
    nrm_r = _sc_norm(dinv, row_r, col_r, w_r)

    W1p = _padw(W1, 128, DP)
    BD2 = _bd4(_padw(W2, DP, DP))
    BD3 = _bd4(_padw(W3, DP, DP))
    I128 = jnp.eye(128, dtype=jnp.float32)
    W4p = _padw(W4, DP, 48)

    h1 = _tc_pre1(features, W1p)
    P = _sc_agg(h1, row_r, col_r, nrm_r, n_pad)
    hp = _tc_step(P.reshape(2 * pr, 128), BD2, _bt4(b1))
    P = _sc_agg(hp.reshape(n_pad, DP), row_r, col_r, nrm_r, n_pad)
    hp = _tc_step(P.reshape(2 * pr, 128), BD2, _bt4(b2))
    P = _sc_agg(hp.reshape(n_pad, DP), row_r, col_r, nrm_r, n_pad)
    hp = _tc_step(P.reshape(2 * pr, 128), BD3, _bt4(b2))
    P = _sc_agg(hp.reshape(n_pad, DP), row_r, col_r, nrm_r, n_pad)
    xp = _tc_step(P.reshape(2 * pr, 128), I128, _bt4(b3))
    P = _sc_agg(xp.reshape(n_pad, DP), row_r, col_r, nrm_r, n_pad)
    return _tc_final(P, W4p, b4[None, :], n, 40)
